# Initial kernel scaffold; baseline (speedup 1.0000x reference)
import jax
import jax.numpy as jnp
from jax.experimental import pallas as pl


def kernel(x, table):
    raise NotImplementedError("write your pallas kernel here")


# speedup vs baseline: 1.4954x; 1.4954x over previous
import functools

import jax
import jax.numpy as jnp
from jax import lax
from jax.experimental import pallas as pl
from jax.experimental.pallas import tpu as pltpu
from jax.experimental.pallas import tpu_sc as plsc

BATCH = 4096
HIST = 200
DIM = 32
NROWS = BATCH * HIST


def _build_gather():
    info = plsc.get_sparse_core_info()
    nc, ns = info.num_cores, info.num_subcores
    nw = nc * ns
    per_w = NROWS // nw
    chunk = 3200
    n_chunks = per_w // chunk

    mesh = plsc.VectorSubcoreMesh(core_axis_name="c", subcore_axis_name="s")

    @functools.partial(
        pl.kernel,
        mesh=mesh,
        compiler_params=pltpu.CompilerParams(use_tc_tiling_on_sc=False),
        out_type=jax.ShapeDtypeStruct((NROWS, DIM), jnp.float32),
        scratch_types=[
            pltpu.VMEM((chunk,), jnp.int32),
            pltpu.VMEM((chunk, DIM), jnp.float32),
            pltpu.SemaphoreType.DMA,
        ],
    )
    def gather(table_hbm, idx_hbm, out_hbm, idx_v, rows_v, sem):
        wid = lax.axis_index("s") * nc + lax.axis_index("c")
        base = wid * per_w

        def body(i, carry):
            off = base + i * chunk
            pltpu.sync_copy(idx_hbm.at[pl.ds(off, chunk)], idx_v)
            pltpu.async_copy(table_hbm.at[idx_v], rows_v, sem).wait()
            pltpu.sync_copy(rows_v, out_hbm.at[pl.ds(off, chunk)])
            return carry

        lax.fori_loop(0, n_chunks, body, 0)

    return gather


_gather = _build_gather()


def kernel(x, table):
    idx = x.reshape(NROWS)
    out = _gather(table, idx)
    return out.reshape(BATCH, HIST, DIM)


# speedup vs baseline: 1.5011x; 1.0038x over previous
import functools

import jax
import jax.numpy as jnp
from jax import lax
from jax.experimental import pallas as pl
from jax.experimental.pallas import tpu as pltpu
from jax.experimental.pallas import tpu_sc as plsc

BATCH = 4096
HIST = 200
DIM = 32
NROWS = BATCH * HIST


def _build_gather():
    info = plsc.get_sparse_core_info()
    nc, ns = info.num_cores, info.num_subcores
    nw = nc * ns
    per_w = NROWS // nw
    chunk = 800
    nbuf = 4
    n_chunks = per_w // chunk

    mesh = plsc.VectorSubcoreMesh(core_axis_name="c", subcore_axis_name="s")

    @functools.partial(
        pl.kernel,
        mesh=mesh,
        compiler_params=pltpu.CompilerParams(use_tc_tiling_on_sc=False),
        out_type=jax.ShapeDtypeStruct((NROWS, DIM), jnp.float32),
        scratch_types=[
            pltpu.VMEM((nbuf, chunk), jnp.int32),
            pltpu.VMEM((nbuf, chunk, DIM), jnp.float32),
            pltpu.SemaphoreType.DMA((nbuf,)),
            pltpu.SemaphoreType.DMA((nbuf,)),
            pltpu.SemaphoreType.DMA((nbuf,)),
        ],
    )
    def gather(table_hbm, idx_hbm, out_hbm, idx_v, rows_v, idx_sem, gat_sem,
               out_sem):
        wid = lax.axis_index("s") * nc + lax.axis_index("c")
        base = wid * per_w

        def idx_copy(i, b):
            return pltpu.async_copy(
                idx_hbm.at[pl.ds(base + i * chunk, chunk)], idx_v.at[b],
                idx_sem.at[b])

        def gat_copy(i, b):
            return pltpu.async_copy(table_hbm.at[idx_v.at[b]], rows_v.at[b],
                                    gat_sem.at[b])

        def out_copy(i, b):
            return pltpu.async_copy(
                rows_v.at[b], out_hbm.at[pl.ds(base + i * chunk, chunk)],
                out_sem.at[b])

        idx_h = {}
        gat_h = {}
        out_h = {}
        for i in range(min(nbuf, n_chunks)):
            idx_h[i] = idx_copy(i, i)
        for i in range(n_chunks):
            b = i % nbuf
            idx_h[i].wait()
            if i >= nbuf:
                out_h[i - nbuf].wait()
            gat_h[i] = gat_copy(i, b)
            if i >= 1:
                p, pb = i - 1, (i - 1) % nbuf
                gat_h[p].wait()
                out_h[p] = out_copy(p, pb)
                if p + nbuf < n_chunks:
                    idx_h[p + nbuf] = idx_copy(p + nbuf, pb)
        last = n_chunks - 1
        gat_h[last].wait()
        out_h[last] = out_copy(last, last % nbuf)
        for i in range(max(0, n_chunks - nbuf), n_chunks):
            out_h[i].wait()

    return gather


_gather = _build_gather()


def kernel(x, table):
    idx = x.reshape(NROWS)
    out = _gather(table, idx)
    return out.reshape(BATCH, HIST, DIM)


# speedup vs baseline: 1.5016x; 1.0004x over previous
import functools

import jax
import jax.numpy as jnp
from jax import lax
from jax.experimental import pallas as pl
from jax.experimental.pallas import tpu as pltpu
from jax.experimental.pallas import tpu_sc as plsc

BATCH = 4096
HIST = 200
DIM = 32
NROWS = BATCH * HIST


def _build_gather():
    info = plsc.get_sparse_core_info()
    nc, ns = info.num_cores, info.num_subcores
    nw = nc * ns
    per_w = NROWS // nw
    chunk = 1280
    nbuf = 2
    n_chunks = per_w // chunk

    mesh = plsc.VectorSubcoreMesh(core_axis_name="c", subcore_axis_name="s")

    @functools.partial(
        pl.kernel,
        mesh=mesh,
        compiler_params=pltpu.CompilerParams(use_tc_tiling_on_sc=False),
        out_type=jax.ShapeDtypeStruct((NROWS, DIM), jnp.float32),
        scratch_types=[
            pltpu.VMEM((per_w,), jnp.int32),
            pltpu.VMEM((nbuf, chunk, DIM), jnp.float32),
            pltpu.SemaphoreType.DMA,
            pltpu.SemaphoreType.DMA((nbuf,)),
            pltpu.SemaphoreType.DMA((nbuf,)),
        ],
    )
    def gather(table_hbm, idx_hbm, out_hbm, idx_all, rows_v, idx_sem, gat_sem,
               out_sem):
        wid = lax.axis_index("s") * nc + lax.axis_index("c")
        base = wid * per_w

        pltpu.async_copy(idx_hbm.at[pl.ds(base, per_w)], idx_all,
                         idx_sem).wait()

        def gat_copy(i, b):
            return pltpu.async_copy(
                table_hbm.at[idx_all.at[pl.ds(i * chunk, chunk)]],
                rows_v.at[b], gat_sem.at[b])

        def out_copy(i, b):
            return pltpu.async_copy(
                rows_v.at[b], out_hbm.at[pl.ds(base + i * chunk, chunk)],
                out_sem.at[b])

        gat_h = {}
        out_h = {}
        for i in range(n_chunks):
            b = i % nbuf
            if i >= nbuf:
                out_h[i - nbuf].wait()
            gat_h[i] = gat_copy(i, b)
            if i >= 1:
                p = i - 1
                gat_h[p].wait()
                out_h[p] = out_copy(p, p % nbuf)
        last = n_chunks - 1
        gat_h[last].wait()
        out_h[last] = out_copy(last, last % nbuf)
        for i in range(max(0, n_chunks - nbuf), n_chunks):
            out_h[i].wait()

    return gather


_gather = _build_gather()


def kernel(x, table):
    idx = x.reshape(NROWS)
    out = _gather(table, idx)
    return out.reshape(BATCH, HIST, DIM)
